# Initial kernel scaffold; baseline (speedup 1.0000x reference)
import jax
import jax.numpy as jnp
from jax.experimental import pallas as pl


def kernel(features, edge_index, edge_type, norm, V0, a0, b0, V1, a1, b1):
    raise NotImplementedError("write your pallas kernel here")


# speedup vs baseline: 7.8045x; 7.8045x over previous
import functools

import jax
import jax.numpy as jnp
from jax import lax
from jax.experimental import pallas as pl
from jax.experimental.pallas import tpu as pltpu
from jax.experimental.pallas import tpu_sc as plsc

N = 10000
E = 320000
D = 128
R = 8
NB = 2

NC = 2
NS = 16
NW = NC * NS
EPW = E // NW
K = 80
NCHUNK = EPW // K
SBC = 25
NSB = NCHUNK // SBC
ROWS_PER_SUB = 624
TAIL_ROWS = N - ROWS_PER_SUB * NS
ZROWS = 48


def _proj_body(a_ref, feat_ref, v_ref, out_ref):
    r = pl.program_id(0)
    w = a_ref[r, 0] * v_ref[0] + a_ref[r, 1] * v_ref[1]
    out_ref[...] = jnp.dot(feat_ref[...], w, preferred_element_type=jnp.float32)


def _eidx_body(et_ref, src_ref, out_ref):
    out_ref[...] = et_ref[...] * N + src_ref[...]


def _epi_body(p_ref, f_ref, b_ref, o_ref):
    h = p_ref[0] + p_ref[1] + b_ref[...]
    o_ref[...] = jnp.maximum(h, 0.0) + f_ref[...]


def _sc_edge_body(hw_hbm, eidx_hbm, dst_hbm, norm_hbm, out_hbm,
                  eidx_v, dst_v, norm_v, rows_v, zbuf, acc_sh, sem):
    cid = lax.axis_index("c")
    sid = lax.axis_index("s")
    wid = sid * NC + cid

    zvec = jnp.zeros((16,), jnp.float32)
    def zero_body(i, _):
        row = i // 8
        col = (i % 8) * 16
        zbuf[row, pl.ds(col, 16)] = zvec
        return 0
    lax.fori_loop(0, ZROWS * (D // 16), zero_body, 0)
    for t in range(ROWS_PER_SUB // ZROWS):
        base = pl.multiple_of(sid * ROWS_PER_SUB + t * ZROWS, 8)
        pltpu.sync_copy(zbuf, acc_sh.at[pl.ds(base, ZROWS)])
    @pl.when(sid == 0)
    def _zero_tail():
        pltpu.sync_copy(zbuf.at[pl.ds(0, TAIL_ROWS)],
                        acc_sh.at[pl.ds(ROWS_PER_SUB * NS, TAIL_ROWS)])
    plsc.subcore_barrier()

    def super_body(sb, _):
        pltpu.sync_copy(eidx_hbm.at[wid, sb], eidx_v)
        pltpu.sync_copy(dst_hbm.at[wid, sb], dst_v)
        pltpu.sync_copy(norm_hbm.at[wid, sb], norm_v)

        def chunk_body(i, _):
            pltpu.async_copy(hw_hbm.at[eidx_v.at[i]], rows_v, sem).wait()

            def scale_body(g, _):
                nv = norm_v[i, pl.ds(g * 16, 16)]
                for l in range(16):
                    w = nv[l]
                    e = g * 16 + l
                    for j in range(D // 16):
                        rows_v[e, pl.ds(j * 16, 16)] = rows_v[e, pl.ds(j * 16, 16)] * w
                return 0
            lax.fori_loop(0, K // 16, scale_body, 0)

            pltpu.sync_copy(rows_v, acc_sh.at[dst_v.at[i]], add=True)
            return 0

        lax.fori_loop(0, SBC, chunk_body, 0)
        return 0

    lax.fori_loop(0, NSB, super_body, 0)
    plsc.subcore_barrier()

    rbase = pl.multiple_of(sid * ROWS_PER_SUB, 8)
    pltpu.sync_copy(acc_sh.at[pl.ds(rbase, ROWS_PER_SUB)],
                    out_hbm.at[cid, pl.ds(rbase, ROWS_PER_SUB)])
    @pl.when(sid == 0)
    def _copy_tail():
        pltpu.sync_copy(acc_sh.at[pl.ds(ROWS_PER_SUB * NS, TAIL_ROWS)],
                        out_hbm.at[cid, pl.ds(ROWS_PER_SUB * NS, TAIL_ROWS)])


def kernel(features, edge_index, edge_type, norm, V0, a0, b0, V1, a1, b1):
    del V0, a0, b0
    src = edge_index[0]
    dst = edge_index[1].reshape(NW, NSB, SBC, K)
    normf = norm.reshape(NW, NSB, SBC, K)

    BN = 1000
    hw = pl.pallas_call(
        _proj_body,
        grid=(R, N // BN),
        in_specs=[
            pl.BlockSpec(memory_space=pltpu.SMEM),
            pl.BlockSpec((BN, D), lambda r, i: (i, 0)),
            pl.BlockSpec((NB, D, D), lambda r, i: (0, 0, 0)),
        ],
        out_specs=pl.BlockSpec((BN, D), lambda r, i: (r * (N // BN) + i, 0)),
        out_shape=jax.ShapeDtypeStruct((R * N, D), jnp.float32),
    )(a1, features, V1)

    EB = 2500
    eidx = pl.pallas_call(
        _eidx_body,
        out_shape=jax.ShapeDtypeStruct((EB, E // EB), jnp.int32),
    )(edge_type.reshape(EB, E // EB), src.reshape(EB, E // EB))
    eidx = eidx.reshape(NW, NSB, SBC, K)

    mesh = plsc.VectorSubcoreMesh(core_axis_name="c", subcore_axis_name="s")
    partials = pl.kernel(
        _sc_edge_body,
        out_type=jax.ShapeDtypeStruct((NC, N, D), jnp.float32),
        mesh=mesh,
        scratch_types=[
            pltpu.VMEM((SBC, K), jnp.int32),
            pltpu.VMEM((SBC, K), jnp.int32),
            pltpu.VMEM((SBC, K), jnp.float32),
            pltpu.VMEM((K, D), jnp.float32),
            pltpu.VMEM((ZROWS, D), jnp.float32),
            pltpu.VMEM_SHARED((N, D), jnp.float32),
            pltpu.SemaphoreType.DMA,
        ],
    )(hw, eidx, dst, normf)

    out = pl.pallas_call(
        _epi_body,
        grid=(N // BN,),
        in_specs=[
            pl.BlockSpec((NC, BN, D), lambda i: (0, i, 0)),
            pl.BlockSpec((BN, D), lambda i: (i, 0)),
            pl.BlockSpec((1, D), lambda i: (0, 0)),
        ],
        out_specs=pl.BlockSpec((BN, D), lambda i: (i, 0)),
        out_shape=jax.ShapeDtypeStruct((N, D), jnp.float32),
    )(partials, features, b1.reshape(1, D))
    return out
